# baseline (speedup 1.0000x reference)
import functools

import jax
import jax.numpy as jnp
from jax import lax
from jax.experimental import pallas as pl
from jax.experimental.pallas import tpu as pltpu
from jax.experimental.pallas import tpu_sc as plsc

N = 10000
DEG = 16
D = 256
DW = D // 2

NC = 2
NS = 16
NW = NC * NS
NODES_PER_TILE = 320
NPAD = NW * NODES_PER_TILE
CHUNK_NODES = 8
CHUNK_ROWS = CHUNK_NODES * DEG
NCHUNKS = NODES_PER_TILE // CHUNK_NODES
WLANES = 16
CB = DW // WLANES

LO_MASK = jnp.int32(0x7FFF)
HI_MASK = jnp.int32(-65536)

MM_BLOCK = 1000


def _mlp_kernel(x_ref, wt_ref, b_ref, o_ref):
    o_ref[...] = jnp.maximum(
        jnp.dot(x_ref[...], wt_ref[...], preferred_element_type=jnp.float32)
        + b_ref[...],
        0.0,
    ).astype(jnp.bfloat16)


def _mlp(features, Wt, b2d):
    return pl.pallas_call(
        _mlp_kernel,
        grid=(N // MM_BLOCK,),
        in_specs=[
            pl.BlockSpec((MM_BLOCK, D), lambda i: (i, 0)),
            pl.BlockSpec((D, D), lambda i: (0, 0)),
            pl.BlockSpec((1, D), lambda i: (0, 0)),
        ],
        out_specs=pl.BlockSpec((MM_BLOCK, D), lambda i: (i, 0)),
        out_shape=jax.ShapeDtypeStruct((N, D), jnp.bfloat16),
    )(features, Wt, b2d)


def _gather_max_body(h_hbm, nbr_hbm, out_hbm, idx_v, bufa, bufb, out_v, sema, semb):
    wid = lax.axis_index("s") * NC + lax.axis_index("c")

    pltpu.sync_copy(nbr_hbm.at[wid], idx_v)

    def start(c, buf, sem):
        pltpu.async_copy(h_hbm.at[idx_v.at[c]], buf, sem)

    def wait(c, buf, sem):
        pltpu.make_async_copy(h_hbm.at[idx_v.at[c]], buf, sem).wait()

    def compute(c, buf):
        def node_body(n, carry):
            for cb in range(CB):
                sl = pl.ds(cb * WLANES, WLANES)
                x = buf[n * DEG, sl]
                acc_hi = x
                acc_lo = x & LO_MASK
                for r in range(1, DEG):
                    x = buf[n * DEG + r, sl]
                    acc_hi = jnp.maximum(acc_hi, x)
                    acc_lo = jnp.maximum(acc_lo, x & LO_MASK)
                out_v[c * CHUNK_NODES + n, sl] = (acc_hi & HI_MASK) | acc_lo
            return carry

        lax.fori_loop(0, CHUNK_NODES, node_body, 0)

    start(0, bufa, sema)
    start(1, bufb, semb)

    def outer(t, carry):
        c0 = 2 * t
        wait(c0, bufa, sema)
        compute(c0, bufa)

        @pl.when(c0 + 2 < NCHUNKS)
        def _():
            start(c0 + 2, bufa, sema)

        c1 = c0 + 1
        wait(c1, bufb, semb)
        compute(c1, bufb)

        @pl.when(c1 + 2 < NCHUNKS)
        def _():
            start(c1 + 2, bufb, semb)

        return carry

    lax.fori_loop(0, NCHUNKS // 2, outer, 0)

    pltpu.sync_copy(out_v, out_hbm.at[pl.ds(wid * NODES_PER_TILE, NODES_PER_TILE)])


@functools.lru_cache(maxsize=1)
def _build_gather_max():
    mesh = plsc.VectorSubcoreMesh(core_axis_name="c", subcore_axis_name="s")
    return pl.kernel(
        _gather_max_body,
        mesh=mesh,
        out_type=jax.ShapeDtypeStruct((NPAD, DW), jnp.int32),
        scratch_types=[
            pltpu.VMEM((NCHUNKS, CHUNK_ROWS), jnp.int32),
            pltpu.VMEM((CHUNK_ROWS, DW), jnp.int32),
            pltpu.VMEM((CHUNK_ROWS, DW), jnp.int32),
            pltpu.VMEM((NODES_PER_TILE, DW), jnp.int32),
            pltpu.SemaphoreType.DMA,
            pltpu.SemaphoreType.DMA,
        ],
    )


def kernel(features, neighbors, W, b):
    h = _mlp(features, W.T, b.reshape(1, D))
    h_view = jax.lax.bitcast_convert_type(h.reshape(N, DW, 2), jnp.int32)
    nbr = jnp.concatenate(
        [neighbors, jnp.zeros((NPAD - N, DEG), jnp.int32)], axis=0
    ).reshape(NW, NCHUNKS, CHUNK_ROWS)
    out = _build_gather_max()(h_view, nbr)
    out_bf16 = jax.lax.bitcast_convert_type(out[:N], jnp.bfloat16).reshape(N, D)
    return out_bf16.astype(jnp.float32)


# speedup vs baseline: 1.4623x; 1.4623x over previous
import functools

import jax
import jax.numpy as jnp
from jax import lax
from jax.experimental import pallas as pl
from jax.experimental.pallas import tpu as pltpu
from jax.experimental.pallas import tpu_sc as plsc

N = 10000
DEG = 16
D = 256
DW = D // 2

NC = 2
NS = 16
NW = NC * NS
NODES_PER_TILE = 320
NPAD = NW * NODES_PER_TILE
CHUNK_NODES = 8
CHUNK_ROWS = CHUNK_NODES * DEG
NCHUNKS = NODES_PER_TILE // CHUNK_NODES
WLANES = 16
CB = DW // WLANES

LO_MASK = jnp.int32(0x7FFF)
HI_MASK = jnp.int32(-65536)

MM_BLOCK = 1000


def _mlp_kernel(x_ref, wt_ref, b_ref, o_ref):
    y = jnp.maximum(
        jnp.dot(x_ref[...], wt_ref[...], preferred_element_type=jnp.float32)
        + b_ref[...],
        0.0,
    )
    lo = jax.lax.bitcast_convert_type(
        y[:, :DW].astype(jnp.bfloat16), jnp.uint16
    ).astype(jnp.uint32)
    hi = jax.lax.bitcast_convert_type(
        y[:, DW:].astype(jnp.bfloat16), jnp.uint16
    ).astype(jnp.uint32)
    o_ref[...] = jax.lax.bitcast_convert_type(lo | (hi << 16), jnp.int32)


def _mlp(features, Wt, b2d):
    return pl.pallas_call(
        _mlp_kernel,
        grid=(N // MM_BLOCK,),
        in_specs=[
            pl.BlockSpec((MM_BLOCK, D), lambda i: (i, 0)),
            pl.BlockSpec((D, D), lambda i: (0, 0)),
            pl.BlockSpec((1, D), lambda i: (0, 0)),
        ],
        out_specs=pl.BlockSpec((MM_BLOCK, DW), lambda i: (i, 0)),
        out_shape=jax.ShapeDtypeStruct((N, DW), jnp.int32),
    )(features, Wt, b2d)


def _gather_max_body(h_hbm, nbr_hbm, out_hbm, idx_v, bufa, bufb, out_v, sema, semb):
    wid = lax.axis_index("s") * NC + lax.axis_index("c")

    pltpu.sync_copy(nbr_hbm.at[wid], idx_v)

    def start(c, buf, sem):
        pltpu.async_copy(h_hbm.at[idx_v.at[c]], buf, sem)

    def wait(c, buf, sem):
        pltpu.make_async_copy(h_hbm.at[idx_v.at[c]], buf, sem).wait()

    def compute(c, buf):
        def node_body(n, carry):
            for cb in range(CB):
                sl = pl.ds(cb * WLANES, WLANES)
                x = buf[n * DEG, sl]
                acc_hi = x
                acc_lo = x & LO_MASK
                for r in range(1, DEG):
                    x = buf[n * DEG + r, sl]
                    acc_hi = jnp.maximum(acc_hi, x)
                    acc_lo = jnp.maximum(acc_lo, x & LO_MASK)
                out_v[c * CHUNK_NODES + n, sl] = (acc_hi & HI_MASK) | acc_lo
            return carry

        lax.fori_loop(0, CHUNK_NODES, node_body, 0)

    start(0, bufa, sema)
    start(1, bufb, semb)

    def outer(t, carry):
        c0 = 2 * t
        wait(c0, bufa, sema)
        compute(c0, bufa)

        @pl.when(c0 + 2 < NCHUNKS)
        def _():
            start(c0 + 2, bufa, sema)

        c1 = c0 + 1
        wait(c1, bufb, semb)
        compute(c1, bufb)

        @pl.when(c1 + 2 < NCHUNKS)
        def _():
            start(c1 + 2, bufb, semb)

        return carry

    lax.fori_loop(0, NCHUNKS // 2, outer, 0)

    pltpu.sync_copy(out_v, out_hbm.at[pl.ds(wid * NODES_PER_TILE, NODES_PER_TILE)])


@functools.lru_cache(maxsize=1)
def _build_gather_max():
    mesh = plsc.VectorSubcoreMesh(core_axis_name="c", subcore_axis_name="s")
    return pl.kernel(
        _gather_max_body,
        mesh=mesh,
        out_type=jax.ShapeDtypeStruct((NPAD, DW), jnp.int32),
        scratch_types=[
            pltpu.VMEM((NCHUNKS, CHUNK_ROWS), jnp.int32),
            pltpu.VMEM((CHUNK_ROWS, DW), jnp.int32),
            pltpu.VMEM((CHUNK_ROWS, DW), jnp.int32),
            pltpu.VMEM((NODES_PER_TILE, DW), jnp.int32),
            pltpu.SemaphoreType.DMA,
            pltpu.SemaphoreType.DMA,
        ],
    )


def kernel(features, neighbors, W, b):
    h_packed = _mlp(features, W.T, b.reshape(1, D))
    nbr = jnp.concatenate(
        [neighbors, jnp.zeros((NPAD - N, DEG), jnp.int32)], axis=0
    ).reshape(NW, NCHUNKS, CHUNK_ROWS)
    out = _build_gather_max()(h_packed, nbr)[:N]
    lo = jax.lax.bitcast_convert_type(
        (out & 0xFFFF).astype(jnp.uint16), jnp.bfloat16
    )
    hi = jax.lax.bitcast_convert_type(
        (out >> 16).astype(jnp.uint16), jnp.bfloat16
    )
    return jnp.concatenate([lo, hi], axis=1).astype(jnp.float32)


# speedup vs baseline: 1.4798x; 1.0119x over previous
import functools

import jax
import jax.numpy as jnp
from jax import lax
from jax.experimental import pallas as pl
from jax.experimental.pallas import tpu as pltpu
from jax.experimental.pallas import tpu_sc as plsc

N = 10000
DEG = 16
D = 256
DW = D // 2

NC = 2
NS = 16
NW = NC * NS
NODES_PER_TILE = 320
NPAD = NW * NODES_PER_TILE
CHUNK_NODES = 8
CHUNK_ROWS = CHUNK_NODES * DEG
NCHUNKS = NODES_PER_TILE // CHUNK_NODES
WLANES = 16
CB = DW // WLANES

LO_MASK = 0x7FFF
HI_MASK = -65536

MM_BLOCK = 1000


def _mlp_kernel(x_ref, wt_ref, b_ref, o_ref):
    y = jnp.maximum(
        jnp.dot(x_ref[...], wt_ref[...], preferred_element_type=jnp.float32)
        + b_ref[...],
        0.0,
    )
    lo = jax.lax.bitcast_convert_type(
        y[:, :DW].astype(jnp.bfloat16), jnp.uint16
    ).astype(jnp.uint32)
    hi = jax.lax.bitcast_convert_type(
        y[:, DW:].astype(jnp.bfloat16), jnp.uint16
    ).astype(jnp.uint32)
    o_ref[...] = jax.lax.bitcast_convert_type(lo | (hi << 16), jnp.int32)


def _mlp(features, Wt, b2d):
    return pl.pallas_call(
        _mlp_kernel,
        grid=(N // MM_BLOCK,),
        in_specs=[
            pl.BlockSpec((MM_BLOCK, D), lambda i: (i, 0)),
            pl.BlockSpec((D, D), lambda i: (0, 0)),
            pl.BlockSpec((1, D), lambda i: (0, 0)),
        ],
        out_specs=pl.BlockSpec((MM_BLOCK, DW), lambda i: (i, 0)),
        out_shape=jax.ShapeDtypeStruct((N, DW), jnp.int32),
    )(features, Wt, b2d)


NBUF = 4


def _gather_max_body(
    h_hbm, nbr_hbm, out_hbm, idx_v, buf0, buf1, buf2, buf3, out_v,
    sem0, sem1, sem2, sem3
):
    bufs = (buf0, buf1, buf2, buf3)
    sems = (sem0, sem1, sem2, sem3)
    wid = lax.axis_index("s") * NC + lax.axis_index("c")

    pltpu.sync_copy(nbr_hbm.at[wid], idx_v)

    def start(c, k):
        pltpu.async_copy(h_hbm.at[idx_v.at[c]], bufs[k], sems[k])

    def wait(c, k):
        pltpu.make_async_copy(h_hbm.at[idx_v.at[c]], bufs[k], sems[k]).wait()

    def compute(c, buf):
        def node_body(n, carry):
            for cb in range(CB):
                sl = pl.ds(cb * WLANES, WLANES)
                x = buf[n * DEG, sl]
                acc_hi = x
                acc_lo = x & LO_MASK
                for r in range(1, DEG):
                    x = buf[n * DEG + r, sl]
                    acc_hi = jnp.maximum(acc_hi, x)
                    acc_lo = jnp.maximum(acc_lo, x & LO_MASK)
                out_v[c * CHUNK_NODES + n, sl] = (acc_hi & HI_MASK) | acc_lo
            return carry

        lax.fori_loop(0, CHUNK_NODES, node_body, 0)

    for k in range(NBUF - 1):
        start(k, k)

    def outer(t, carry):
        base = NBUF * t
        for k in range(NBUF):
            c = base + k

            @pl.when(c + NBUF - 1 < NCHUNKS)
            def _():
                start(c + NBUF - 1, (k + NBUF - 1) % NBUF)

            wait(c, k)
            compute(c, bufs[k])
        return carry

    lax.fori_loop(0, NCHUNKS // NBUF, outer, 0)

    pltpu.sync_copy(out_v, out_hbm.at[pl.ds(wid * NODES_PER_TILE, NODES_PER_TILE)])


@functools.lru_cache(maxsize=1)
def _build_gather_max():
    mesh = plsc.VectorSubcoreMesh(core_axis_name="c", subcore_axis_name="s")
    return pl.kernel(
        _gather_max_body,
        mesh=mesh,
        out_type=jax.ShapeDtypeStruct((NPAD, DW), jnp.int32),
        scratch_types=(
            [pltpu.VMEM((NCHUNKS, CHUNK_ROWS), jnp.int32)]
            + [pltpu.VMEM((CHUNK_ROWS, DW), jnp.int32)] * NBUF
            + [pltpu.VMEM((NODES_PER_TILE, DW), jnp.int32)]
            + [pltpu.SemaphoreType.DMA] * NBUF
        ),
    )


def kernel(features, neighbors, W, b):
    h_packed = _mlp(features, W.T, b.reshape(1, D))
    nbr = jnp.concatenate(
        [neighbors, jnp.zeros((NPAD - N, DEG), jnp.int32)], axis=0
    ).reshape(NW, NCHUNKS, CHUNK_ROWS)
    out = _build_gather_max()(h_packed, nbr)[:N]
    lo = jax.lax.bitcast_convert_type(
        (out & 0xFFFF).astype(jnp.uint16), jnp.bfloat16
    )
    hi = jax.lax.bitcast_convert_type(
        (out >> 16).astype(jnp.uint16), jnp.bfloat16
    )
    return jnp.concatenate([lo, hi], axis=1).astype(jnp.float32)


# speedup vs baseline: 1.4798x; 1.0000x over previous
import functools

import jax
import jax.numpy as jnp
from jax import lax
from jax.experimental import pallas as pl
from jax.experimental.pallas import tpu as pltpu
from jax.experimental.pallas import tpu_sc as plsc

N = 10000
DEG = 16
D = 256
DW = D // 2

NC = 2
NS = 16
NW = NC * NS
NODES_PER_TILE = 320
NPAD = NW * NODES_PER_TILE
CHUNK_NODES = 8
CHUNK_ROWS = CHUNK_NODES * DEG
NCHUNKS = NODES_PER_TILE // CHUNK_NODES
NBUF = 4
WLANES = 16
CB = DW // WLANES

LO_MASK = 0x7FFF
HI_MASK = -65536

MM_BLOCK = 1000


def _mlp_kernel(x_ref, wt_ref, b_ref, o_ref):
    y = jnp.maximum(
        jnp.dot(x_ref[...], wt_ref[...], preferred_element_type=jnp.float32)
        + b_ref[...],
        0.0,
    )
    lo = jax.lax.bitcast_convert_type(
        y[:, :DW].astype(jnp.bfloat16), jnp.uint16
    ).astype(jnp.uint32)
    hi = jax.lax.bitcast_convert_type(
        y[:, DW:].astype(jnp.bfloat16), jnp.uint16
    ).astype(jnp.uint32)
    o_ref[...] = jax.lax.bitcast_convert_type(lo | (hi << 16), jnp.int32)


def _mlp(features, Wt, b2d):
    return pl.pallas_call(
        _mlp_kernel,
        grid=(N // MM_BLOCK,),
        in_specs=[
            pl.BlockSpec((MM_BLOCK, D), lambda i: (i, 0)),
            pl.BlockSpec((D, D), lambda i: (0, 0)),
            pl.BlockSpec((1, D), lambda i: (0, 0)),
        ],
        out_specs=pl.BlockSpec((MM_BLOCK, DW), lambda i: (i, 0)),
        out_shape=jax.ShapeDtypeStruct((N, DW), jnp.int32),
    )(features, Wt, b2d)


def _gather_max_body(
    h_hbm, nbr_hbm, out_hbm, idx_v, buf0, buf1, buf2, buf3, out_v,
    sem0, sem1, sem2, sem3
):
    bufs = (buf0, buf1, buf2, buf3)
    sems = (sem0, sem1, sem2, sem3)
    wid = lax.axis_index("s") * NC + lax.axis_index("c")

    pltpu.sync_copy(nbr_hbm.at[wid], idx_v)

    def start(c, k):
        pltpu.async_copy(h_hbm.at[idx_v.at[c]], bufs[k], sems[k])

    def wait(c, k):
        pltpu.make_async_copy(h_hbm.at[idx_v.at[c]], bufs[k], sems[k]).wait()

    def compute(c, buf):
        def node_body(n, carry):
            for cb in range(CB):
                sl = pl.ds(cb * WLANES, WLANES)
                x = buf[n * DEG, sl]
                acc_hi = x
                acc_lo = x & LO_MASK
                for r in range(1, DEG):
                    x = buf[n * DEG + r, sl]
                    acc_hi = jnp.maximum(acc_hi, x)
                    acc_lo = jnp.maximum(acc_lo, x & LO_MASK)
                out_v[c * CHUNK_NODES + n, sl] = (acc_hi & HI_MASK) | acc_lo
            return carry

        lax.fori_loop(0, CHUNK_NODES, node_body, 0)

    for k in range(NBUF - 1):
        start(k, k)

    def outer(t, carry):
        base = NBUF * t
        for k in range(NBUF):
            c = base + k

            @pl.when(c + NBUF - 1 < NCHUNKS)
            def _():
                start(c + NBUF - 1, (k + NBUF - 1) % NBUF)

            wait(c, k)
            compute(c, bufs[k])
        return carry

    lax.fori_loop(0, NCHUNKS // NBUF, outer, 0)

    pltpu.sync_copy(out_v, out_hbm.at[pl.ds(wid * NODES_PER_TILE, NODES_PER_TILE)])


@functools.lru_cache(maxsize=1)
def _build_gather_max():
    mesh = plsc.VectorSubcoreMesh(core_axis_name="c", subcore_axis_name="s")
    return pl.kernel(
        _gather_max_body,
        mesh=mesh,
        out_type=jax.ShapeDtypeStruct((NPAD, DW), jnp.int32),
        scratch_types=(
            [pltpu.VMEM((NCHUNKS, CHUNK_ROWS), jnp.int32)]
            + [pltpu.VMEM((CHUNK_ROWS, DW), jnp.int32)] * NBUF
            + [pltpu.VMEM((NODES_PER_TILE, DW), jnp.int32)]
            + [pltpu.SemaphoreType.DMA] * NBUF
        ),
    )


def kernel(features, neighbors, W, b):
    h_packed = _mlp(features, W.T, b.reshape(1, D))
    nbr = jnp.concatenate(
        [neighbors, jnp.zeros((NPAD - N, DEG), jnp.int32)], axis=0
    ).reshape(NW, NCHUNKS, CHUNK_ROWS)
    out = _build_gather_max()(h_packed, nbr)[:N]
    lo = jax.lax.bitcast_convert_type(
        (out & 0xFFFF).astype(jnp.uint16), jnp.bfloat16
    )
    hi = jax.lax.bitcast_convert_type(
        (out >> 16).astype(jnp.uint16), jnp.bfloat16
    )
    return jnp.concatenate([lo, hi], axis=1).astype(jnp.float32)
